# Initial kernel scaffold; baseline (speedup 1.0000x reference)
import jax
import jax.numpy as jnp
from jax.experimental import pallas as pl


def kernel(x, sub_nodes, adj, W, b):
    raise NotImplementedError("write your pallas kernel here")


# speedup vs baseline: 1.8024x; 1.8024x over previous
import jax
import jax.numpy as jnp
from jax.experimental import pallas as pl
from jax.experimental.pallas import tpu as pltpu
from jax.experimental.pallas import tpu_sc as plsc

_N = 16
_D = 128
_GATHER_WINDOW = 128
_B = 256
_G = 16


def _sc_gather(x, flat_idx):
    n = flat_idx.shape[1]
    d = x.shape[1]
    mesh = plsc.VectorSubcoreMesh(core_axis_name="core", subcore_axis_name="subcore")

    @pl.kernel(out_type=jax.ShapeDtypeStruct((n, d), x.dtype), mesh=mesh)
    def gather_kernel(x_hbm, i_hbm, o_hbm):
        def body(i_vmem, o_vmem):
            pltpu.sync_copy(x_hbm.at[i_vmem.at[0]], o_vmem)

        pltpu.emit_pipeline(
            body,
            grid=(n // _GATHER_WINDOW,),
            in_specs=[pl.BlockSpec((1, _GATHER_WINDOW), index_map=lambda i: (0, i))],
            out_specs=[pl.BlockSpec((_GATHER_WINDOW, d), index_map=lambda i: (i, 0))],
            core_axis_name=("core", "subcore"),
            dimension_semantics=(pltpu.PARALLEL,),
        )(i_hbm, o_hbm)

    return gather_kernel(x, flat_idx)


def _tc_block(g_ref, adj_ref, w_ref, b_ref, o_ref):
    gn = _G * _N
    y = jnp.dot(g_ref[...], w_ref[...], preferred_element_type=jnp.float32)
    a = adj_ref[...].reshape(_B * _N, _N)
    a = a / (jnp.sum(a, axis=1, keepdims=True) + 1e-8)
    rows = jax.lax.broadcasted_iota(jnp.int32, (gn, gn), 0)
    cols = jax.lax.broadcasted_iota(jnp.int32, (gn, gn), 1)
    mask = (rows // _N == cols // _N).astype(jnp.float32)
    b_row = b_ref[...]
    for g in range(_B // _G):
        sl = slice(g * gn, (g + 1) * gn)
        a_g = a[sl]
        bd = jnp.concatenate([a_g] * _G, axis=1) * mask
        z = jnp.dot(bd, y[sl], preferred_element_type=jnp.float32) + b_row
        h = jnp.where(z > 0, z, jnp.exp(z) - 1.0)
        h3 = h.reshape(_G, _N, _D)
        colsum = jnp.sum(h3, axis=1)
        mean = jnp.sum(colsum, axis=1, keepdims=True) / (_N * _D)
        sumsq = jnp.sum(jnp.sum(h3 * h3, axis=1), axis=1, keepdims=True)
        var = sumsq / (_N * _D) - mean * mean
        o_ref[g * _G:(g + 1) * _G, :] = (colsum - _N * mean) * jax.lax.rsqrt(var + 1e-5)


def kernel(x, sub_nodes, adj, W, b):
    n_subs = sub_nodes.shape[0]
    flat_idx = sub_nodes.reshape(1, n_subs * _N)
    gathered = _sc_gather(x, flat_idx)
    b2 = b.reshape(1, _D)
    out = pl.pallas_call(
        _tc_block,
        grid=(n_subs // _B,),
        in_specs=[
            pl.BlockSpec((_B * _N, _D), lambda i: (i, 0)),
            pl.BlockSpec((_B, _N, _N), lambda i: (i, 0, 0)),
            pl.BlockSpec((_D, _D), lambda i: (0, 0)),
            pl.BlockSpec((1, _D), lambda i: (0, 0)),
        ],
        out_specs=pl.BlockSpec((_B, _D), lambda i: (i, 0)),
        out_shape=jax.ShapeDtypeStruct((n_subs, _D), jnp.float32),
    )(gathered, adj, W, b2)
    return out


# speedup vs baseline: 3.6065x; 2.0010x over previous
import jax
import jax.numpy as jnp
from jax.experimental import pallas as pl
from jax.experimental.pallas import tpu as pltpu
from jax.experimental.pallas import tpu_sc as plsc

_N = 16
_D = 128
_GATHER_WINDOW = 128
_B = 256
_G = 16


def _sc_gather(x, flat_idx):
    n = flat_idx.shape[1]
    d = x.shape[1]
    mesh = plsc.VectorSubcoreMesh(core_axis_name="core", subcore_axis_name="subcore")

    @pl.kernel(out_type=jax.ShapeDtypeStruct((n, d), x.dtype), mesh=mesh)
    def gather_kernel(x_hbm, i_hbm, o_hbm):
        def body(i_vmem, o_vmem):
            pltpu.sync_copy(x_hbm.at[i_vmem.at[0]], o_vmem)

        pltpu.emit_pipeline(
            body,
            grid=(n // _GATHER_WINDOW,),
            in_specs=[pl.BlockSpec((1, _GATHER_WINDOW), index_map=lambda i: (0, i))],
            out_specs=[pl.BlockSpec((_GATHER_WINDOW, d), index_map=lambda i: (i, 0))],
            core_axis_name=("core", "subcore"),
            dimension_semantics=(pltpu.PARALLEL,),
        )(i_hbm, o_hbm)

    return gather_kernel(x, flat_idx)


_BF = jnp.bfloat16


def _dot(a, b):
    return jnp.dot(a, b, preferred_element_type=jnp.float32)


def _tc_block(g_ref, adj_ref, w_ref, b_ref, o_ref):
    gn = _G * _N
    ng = _B // _G
    rows_tot = _B * _N
    yb = _dot(g_ref[...].astype(_BF), w_ref[...]).astype(_BF)
    a = adj_ref[...].reshape(rows_tot, _N)
    rinv = 1.0 / (_dot(a, jnp.ones((_N, _D), _BF)) + 1e-8)
    tq = jax.lax.broadcasted_iota(jnp.int32, (_N, gn), 1)
    tj = jax.lax.broadcasted_iota(jnp.int32, (_N, gn), 0)
    tile = (tq % _N == tj).astype(_BF)
    rows = jax.lax.broadcasted_iota(jnp.int32, (gn, gn), 0)
    cols = jax.lax.broadcasted_iota(jnp.int32, (gn, gn), 1)
    mask = (rows // _N == cols // _N).astype(_BF)
    mask_all = jnp.concatenate([mask] * ng, axis=0)
    bd_all = _dot(a, tile).astype(_BF) * mask_all
    z = jnp.concatenate(
        [_dot(bd_all[g * gn:(g + 1) * gn], yb[g * gn:(g + 1) * gn])
         for g in range(ng)], axis=0)
    z = z * rinv + b_ref[...]
    h = jnp.where(z > 0, z, jnp.exp(z) - 1.0)
    hb = h.astype(_BF)
    sp = jax.lax.broadcasted_iota(jnp.int32, (_G, gn), 1)
    ss = jax.lax.broadcasted_iota(jnp.int32, (_G, gn), 0)
    seg = (sp // _N == ss).astype(_BF)
    colsum = jnp.concatenate(
        [_dot(seg, hb[g * gn:(g + 1) * gn]) for g in range(ng)], axis=0)
    sumsq = jnp.concatenate(
        [_dot(seg, hb[g * gn:(g + 1) * gn] * hb[g * gn:(g + 1) * gn])
         for g in range(ng)], axis=0)
    ones_dd = jnp.ones((_D, _D), _BF)
    inv_nd = 1.0 / (_N * _D)
    mean = _dot(colsum.astype(_BF), ones_dd) * inv_nd
    var = _dot(sumsq.astype(_BF), ones_dd) * inv_nd - mean * mean
    o_ref[...] = (colsum - _N * mean) * jax.lax.rsqrt(var + 1e-5)


def kernel(x, sub_nodes, adj, W, b):
    n_subs = sub_nodes.shape[0]
    flat_idx = sub_nodes.reshape(1, n_subs * _N)
    gathered = _sc_gather(x, flat_idx)
    adj = adj.astype(_BF)
    W = W.astype(_BF)
    b2 = b.reshape(1, _D)
    out = pl.pallas_call(
        _tc_block,
        grid=(n_subs // _B,),
        in_specs=[
            pl.BlockSpec((_B * _N, _D), lambda i: (i, 0)),
            pl.BlockSpec((_B, _N, _N), lambda i: (i, 0, 0)),
            pl.BlockSpec((_D, _D), lambda i: (0, 0)),
            pl.BlockSpec((1, _D), lambda i: (0, 0)),
        ],
        out_specs=pl.BlockSpec((_B, _D), lambda i: (i, 0)),
        out_shape=jax.ShapeDtypeStruct((n_subs, _D), jnp.float32),
    )(gathered, adj, W, b2)
    return out
